# baseline (speedup 1.0000x reference)
import functools

import jax
import jax.numpy as jnp
from jax import lax
from jax.experimental import pallas as pl
from jax.experimental.pallas import tpu as pltpu
from jax.experimental.pallas import tpu_sc as plsc

N = 10000
E = 320000
C = 128
NC = 2
NS = 16
NT = NC * NS
K = 80
EPT = E // NT
EPC = E // NC
EPCT = EPC // NS
CPT = EPCT // K
RPT = N // NS
ZR = 25
BN = 1000
GRID = N // BN

_MESH = plsc.VectorSubcoreMesh(
    core_axis_name="c", subcore_axis_name="s", num_cores=NC, num_subcores=NS)


def _deg_body(dst_hbm, w_hbm, out_hbm, dbuf, wbuf, deg_v):
    c = lax.axis_index("c")
    s = lax.axis_index("s")
    wid = c * NS + s
    base = wid * EPT

    @pl.loop(0, N // 16)
    def _zero(j):
        deg_v[pl.ds(j * 16, 16)] = jnp.zeros((16,), jnp.float32)

    pltpu.sync_copy(dst_hbm.at[pl.ds(base, EPT)], dbuf)
    pltpu.sync_copy(w_hbm.at[pl.ds(base, EPT)], wbuf)

    @pl.loop(0, EPT // 16)
    def _acc(j):
        d = dbuf[pl.ds(j * 16, 16)]
        v = wbuf[pl.ds(j * 16, 16)]
        plsc.addupdate_scatter(deg_v, [d], v)

    pltpu.sync_copy(deg_v, out_hbm.at[wid])


_deg_kernel = functools.partial(
    pl.kernel,
    out_type=jax.ShapeDtypeStruct((NT, N), jnp.float32),
    mesh=_MESH,
    compiler_params=pltpu.CompilerParams(
        needs_layout_passes=False, use_tc_tiling_on_sc=False),
    scratch_types=[
        pltpu.VMEM((EPT,), jnp.int32),
        pltpu.VMEM((EPT,), jnp.float32),
        pltpu.VMEM((N,), jnp.float32),
    ],
)(_deg_body)


def _scat_body(xs_hbm, src_hbm, dst_hbm, w_hbm, out_hbm,
               srcbuf, dstbuf, wbuf, rows_a, rows_b, acc,
               gsem, ssem_a, ssem_b):
    c = lax.axis_index("c")
    s = lax.axis_index("s")
    wid = c * NS + s

    pltpu.sync_copy(src_hbm.at[wid], srcbuf)
    pltpu.sync_copy(dst_hbm.at[wid], dstbuf)
    pltpu.sync_copy(w_hbm.at[wid], wbuf)

    @pl.loop(0, K)
    def _zrb(r):
        for cc in range(C // 16):
            rows_b[r, pl.ds(cc * 16, 16)] = jnp.zeros((16,), jnp.float32)

    @pl.loop(0, RPT // K)
    def _zc(j):
        pltpu.sync_copy(rows_b, acc.at[pl.ds(s * RPT + j * K, K)])

    pltpu.sync_copy(rows_b.at[pl.ds(0, RPT - (RPT // K) * K)],
                    acc.at[pl.ds(s * RPT + (RPT // K) * K,
                                 RPT - (RPT // K) * K)])

    plsc.subcore_barrier()

    def scale(rows, i):
        @pl.loop(0, K // 16)
        def _grp(g):
            wv = wbuf[i, pl.ds(g * 16, 16)]
            for j in range(16):
                e = g * 16 + j
                bc = jnp.full((16,), wv[j], jnp.float32)
                for cc in range(C // 16):
                    sl = pl.ds(cc * 16, 16)
                    rows[e, sl] = rows[e, sl] * bc

    my_xs = xs_hbm.at[c]

    def gather_start(i, rows):
        pltpu.async_copy(my_xs.at[srcbuf.at[i].at[pl.ds(0, K // 2)]],
                         rows.at[pl.ds(0, K // 2)], gsem)
        pltpu.async_copy(my_xs.at[srcbuf.at[i].at[pl.ds(K // 2, K // 2)]],
                         rows.at[pl.ds(K // 2, K // 2)], gsem)

    def gather_wait(rows):
        pltpu.make_async_copy(my_xs.at[srcbuf.at[0].at[pl.ds(0, K // 2)]],
                              rows.at[pl.ds(0, K // 2)], gsem).wait()
        pltpu.make_async_copy(my_xs.at[srcbuf.at[0].at[pl.ds(0, K // 2)]],
                              rows.at[pl.ds(K // 2, K // 2)], gsem).wait()

    def scat_start(rows, i, sem):
        pltpu.async_copy(rows, acc.at[dstbuf.at[i]], sem, add=True)

    def scat_wait(rows, sem):
        pltpu.make_async_copy(rows, acc.at[dstbuf.at[0]], sem).wait()

    gather_start(0, rows_a)
    scat_start(rows_b, 0, ssem_b)

    @pl.loop(0, (CPT - 1) // 2)
    def _pair(p):
        i0 = 2 * p
        gather_wait(rows_a)
        scat_wait(rows_b, ssem_b)
        gather_start(i0 + 1, rows_b)
        scale(rows_a, i0)
        scat_start(rows_a, i0, ssem_a)
        gather_wait(rows_b)
        scat_wait(rows_a, ssem_a)
        gather_start(i0 + 2, rows_a)
        scale(rows_b, i0 + 1)
        scat_start(rows_b, i0 + 1, ssem_b)

    gather_wait(rows_a)
    scat_wait(rows_b, ssem_b)
    scale(rows_a, CPT - 1)
    scat_start(rows_a, CPT - 1, ssem_a)
    scat_wait(rows_a, ssem_a)

    plsc.subcore_barrier()
    pltpu.sync_copy(acc.at[pl.ds(s * RPT, RPT)],
                    out_hbm.at[c].at[pl.ds(s * RPT, RPT)])


_scat_kernel = functools.partial(
    pl.kernel,
    out_type=jax.ShapeDtypeStruct((NC, N, C), jnp.float32),
    mesh=_MESH,
    compiler_params=pltpu.CompilerParams(
        needs_layout_passes=False, use_tc_tiling_on_sc=False),
    scratch_types=[
        pltpu.VMEM((CPT, K), jnp.int32),
        pltpu.VMEM((CPT, K), jnp.int32),
        pltpu.VMEM((CPT, K), jnp.float32),
        pltpu.VMEM((K, C), jnp.float32),
        pltpu.VMEM((K, C), jnp.float32),
        pltpu.VMEM_SHARED((N, C), jnp.float32),
        pltpu.SemaphoreType.DMA,
        pltpu.SemaphoreType.DMA,
        pltpu.SemaphoreType.DMA,
    ],
)(_scat_body)


def _dinv_of(dp_ref):
    return lax.rsqrt(jnp.sum(dp_ref[0], axis=0) + 1.0)[:, None]


def _prep_body(dp_ref, x_ref, xs_ref, xs2_ref):
    xs = _dinv_of(dp_ref) * x_ref[...]
    xs_ref[...] = xs
    xs2_ref[...] = xs


def _prep_call(deg_part, x):
    row = pl.BlockSpec((BN, C), lambda i: (i, 0))
    return pl.pallas_call(
        _prep_body,
        grid=(GRID,),
        in_specs=[pl.BlockSpec((1, NT, BN), lambda i: (i, 0, 0)), row],
        out_specs=[row, row],
        out_shape=[jax.ShapeDtypeStruct((N, C), jnp.float32),
                   jax.ShapeDtypeStruct((N, C), jnp.float32)],
    )(deg_part, x)


def _cell_body(final, dp_ref, xs_ref, p0_ref, p1_ref,
               wz_ref, lz_ref, bz_ref, wh_ref, lh_ref, bh_ref,
               lw_ref, lb_ref, out_ref, out2_ref):
    dinv = _dinv_of(dp_ref)
    xs = xs_ref[...]
    agg = dinv * (p0_ref[...] + p1_ref[...] + xs)
    dot = functools.partial(jnp.dot, preferred_element_type=jnp.float32)
    wzf = dot(wz_ref[...], lz_ref[...])
    whf = dot(wh_ref[...], lh_ref[...])
    z = jax.nn.sigmoid(dot(agg, wzf) + bz_ref[...])
    ht = jnp.tanh(dot(agg, whf) + bh_ref[...])
    h = (1.0 - z) * ht
    if final:
        out_ref[...] = dot(h, lw_ref[...]) + lb_ref[...]
        out2_ref[...] = jnp.zeros_like(out2_ref)
    else:
        xs2 = dinv * h
        out_ref[...] = xs2
        out2_ref[...] = xs2


def _cell_call(final, deg_part, xs, p0, p1, p, lin_w, lin_b):
    row = pl.BlockSpec((BN, C), lambda i: (i, 0))
    full = pl.BlockSpec((C, C), lambda i: (0, 0))
    vec = pl.BlockSpec((1, C), lambda i: (0, 0))
    lz = p['lin_z_W'][:C]
    lh = p['lin_h_W'][:C]
    bz = (p['conv_z_b'] @ lz + p['lin_z_b'])[None, :]
    bh = (p['conv_h_b'] @ lh + p['lin_h_b'])[None, :]
    return pl.pallas_call(
        functools.partial(_cell_body, final),
        grid=(GRID,),
        in_specs=[pl.BlockSpec((1, NT, BN), lambda i: (i, 0, 0)), row, row, row,
                  full, full, vec, full, full, vec, full, vec],
        out_specs=[row, row],
        out_shape=[jax.ShapeDtypeStruct((N, C), jnp.float32),
                   jax.ShapeDtypeStruct((N, C), jnp.float32)],
    )(deg_part, xs, p0, p1,
      p['conv_z_W'], lz, bz, p['conv_h_W'], lh, bh,
      lin_w, lin_b[None, :])


def kernel(x, edge_index, edge_weight, params):
    src = edge_index[0].astype(jnp.int32)
    dst = edge_index[1].astype(jnp.int32)
    w = edge_weight.astype(jnp.float32)
    src3d = src.reshape(NT, CPT, K)
    dst3d = dst.reshape(NT, CPT, K)
    w3d = w.reshape(NT, CPT, K)

    deg_part = _deg_kernel(dst, w)
    deg_part = deg_part.reshape(NT, GRID, BN).transpose(1, 0, 2)
    xs1, xs1c = _prep_call(deg_part, x)
    parts1 = _scat_kernel(jnp.stack([xs1, xs1c]), src3d, dst3d, w3d)
    zero_w = jnp.zeros((C, C), jnp.float32)
    zero_b = jnp.zeros((C,), jnp.float32)
    xs2, xs2c = _cell_call(False, deg_part, xs1, parts1[0], parts1[1],
                           params['tgcn1'], zero_w, zero_b)
    parts2 = _scat_kernel(jnp.stack([xs2, xs2c]), src3d, dst3d, w3d)
    out, _ = _cell_call(True, deg_part, xs2, parts2[0], parts2[1],
                        params['tgcn2'], params['lin_W'], params['lin_b'])
    return out


# speedup vs baseline: 1.0479x; 1.0479x over previous
import functools

import jax
import jax.numpy as jnp
from jax import lax
from jax.experimental import pallas as pl
from jax.experimental.pallas import tpu as pltpu
from jax.experimental.pallas import tpu_sc as plsc

N = 10000
E = 320000
C = 128
NC = 2
NS = 16
NT = NC * NS
K = 80
EPT = E // NT
EPC = E // NC
EPCT = EPC // NS
CPT = EPCT // K
RPT = N // NS
ZR = 25
BN = 1000
GRID = N // BN

_MESH = plsc.VectorSubcoreMesh(
    core_axis_name="c", subcore_axis_name="s", num_cores=NC, num_subcores=NS)


def _deg_body(dst_hbm, w_hbm, out_hbm, dbuf, wbuf, deg_v):
    c = lax.axis_index("c")
    s = lax.axis_index("s")
    wid = c * NS + s
    base = wid * EPT

    @pl.loop(0, N // 16)
    def _zero(j):
        deg_v[pl.ds(j * 16, 16)] = jnp.zeros((16,), jnp.float32)

    pltpu.sync_copy(dst_hbm.at[pl.ds(base, EPT)], dbuf)
    pltpu.sync_copy(w_hbm.at[pl.ds(base, EPT)], wbuf)

    @pl.loop(0, EPT // 16)
    def _acc(j):
        d = dbuf[pl.ds(j * 16, 16)]
        v = wbuf[pl.ds(j * 16, 16)]
        plsc.addupdate_scatter(deg_v, [d], v)

    pltpu.sync_copy(deg_v, out_hbm.at[wid])


_deg_kernel = functools.partial(
    pl.kernel,
    out_type=jax.ShapeDtypeStruct((NT, N), jnp.float32),
    mesh=_MESH,
    compiler_params=pltpu.CompilerParams(
        needs_layout_passes=False, use_tc_tiling_on_sc=False),
    scratch_types=[
        pltpu.VMEM((EPT,), jnp.int32),
        pltpu.VMEM((EPT,), jnp.float32),
        pltpu.VMEM((N,), jnp.float32),
    ],
)(_deg_body)


def _scat_body(xs_hbm, src_hbm, dst_hbm, w_hbm, out_hbm,
               srcbuf, dstbuf, wbuf, rows_a, rows_b, acc,
               gsem, ssem_a, ssem_b):
    c = lax.axis_index("c")
    s = lax.axis_index("s")
    wid = c * NS + s

    pltpu.sync_copy(src_hbm.at[wid], srcbuf)
    pltpu.sync_copy(dst_hbm.at[wid], dstbuf)
    pltpu.sync_copy(w_hbm.at[wid], wbuf)

    @pl.loop(0, K)
    def _zrb(r):
        for cc in range(C // 16):
            rows_b[r, pl.ds(cc * 16, 16)] = jnp.zeros((16,), jnp.float32)

    @pl.loop(0, RPT // K)
    def _zc(j):
        pltpu.sync_copy(rows_b, acc.at[pl.ds(s * RPT + j * K, K)])

    pltpu.sync_copy(rows_b.at[pl.ds(0, RPT - (RPT // K) * K)],
                    acc.at[pl.ds(s * RPT + (RPT // K) * K,
                                 RPT - (RPT // K) * K)])

    plsc.subcore_barrier()

    def scale(rows, i):
        @pl.loop(0, K // 16)
        def _grp(g):
            wv = wbuf[i, pl.ds(g * 16, 16)]
            for j in range(16):
                e = g * 16 + j
                bc = jnp.full((16,), wv[j], jnp.float32)
                for cc in range(C // 16):
                    sl = pl.ds(cc * 16, 16)
                    rows[e, sl] = rows[e, sl] * bc

    def gather_start(i, rows):
        pltpu.async_copy(xs_hbm.at[srcbuf.at[i].at[pl.ds(0, K // 2)]],
                         rows.at[pl.ds(0, K // 2)], gsem)
        pltpu.async_copy(xs_hbm.at[srcbuf.at[i].at[pl.ds(K // 2, K // 2)]],
                         rows.at[pl.ds(K // 2, K // 2)], gsem)

    def gather_wait(rows):
        pltpu.make_async_copy(xs_hbm.at[srcbuf.at[0].at[pl.ds(0, K // 2)]],
                              rows.at[pl.ds(0, K // 2)], gsem).wait()
        pltpu.make_async_copy(xs_hbm.at[srcbuf.at[0].at[pl.ds(0, K // 2)]],
                              rows.at[pl.ds(K // 2, K // 2)], gsem).wait()

    def scat_start(rows, i, sem):
        pltpu.async_copy(rows, acc.at[dstbuf.at[i]], sem, add=True)

    def scat_wait(rows, sem):
        pltpu.make_async_copy(rows, acc.at[dstbuf.at[0]], sem).wait()

    gather_start(0, rows_a)
    scat_start(rows_b, 0, ssem_b)

    @pl.loop(0, (CPT - 1) // 2)
    def _pair(p):
        i0 = 2 * p
        gather_wait(rows_a)
        scat_wait(rows_b, ssem_b)
        gather_start(i0 + 1, rows_b)
        scale(rows_a, i0)
        scat_start(rows_a, i0, ssem_a)
        gather_wait(rows_b)
        scat_wait(rows_a, ssem_a)
        gather_start(i0 + 2, rows_a)
        scale(rows_b, i0 + 1)
        scat_start(rows_b, i0 + 1, ssem_b)

    gather_wait(rows_a)
    scat_wait(rows_b, ssem_b)
    scale(rows_a, CPT - 1)
    scat_start(rows_a, CPT - 1, ssem_a)
    scat_wait(rows_a, ssem_a)

    plsc.subcore_barrier()
    pltpu.sync_copy(acc.at[pl.ds(s * RPT, RPT)],
                    out_hbm.at[c].at[pl.ds(s * RPT, RPT)])


_scat_kernel = functools.partial(
    pl.kernel,
    out_type=jax.ShapeDtypeStruct((NC, N, C), jnp.float32),
    mesh=_MESH,
    compiler_params=pltpu.CompilerParams(
        needs_layout_passes=False, use_tc_tiling_on_sc=False),
    scratch_types=[
        pltpu.VMEM((CPT, K), jnp.int32),
        pltpu.VMEM((CPT, K), jnp.int32),
        pltpu.VMEM((CPT, K), jnp.float32),
        pltpu.VMEM((K, C), jnp.float32),
        pltpu.VMEM((K, C), jnp.float32),
        pltpu.VMEM_SHARED((N, C), jnp.float32),
        pltpu.SemaphoreType.DMA,
        pltpu.SemaphoreType.DMA,
        pltpu.SemaphoreType.DMA,
    ],
)(_scat_body)


def _dinv_of(dp_ref):
    return lax.rsqrt(jnp.sum(dp_ref[0], axis=0) + 1.0)[:, None]


def _prep_body(dp_ref, x_ref, xs_ref):
    xs_ref[...] = _dinv_of(dp_ref) * x_ref[...]


def _prep_call(deg_part, x):
    row = pl.BlockSpec((BN, C), lambda i: (i, 0))
    return pl.pallas_call(
        _prep_body,
        grid=(GRID,),
        in_specs=[pl.BlockSpec((1, NT, BN), lambda i: (i, 0, 0)), row],
        out_specs=row,
        out_shape=jax.ShapeDtypeStruct((N, C), jnp.float32),
    )(deg_part, x)


def _cell_body(final, dp_ref, xs_ref, p0_ref, p1_ref,
               wz_ref, lz_ref, bz_ref, wh_ref, lh_ref, bh_ref,
               lw_ref, lb_ref, out_ref):
    dinv = _dinv_of(dp_ref)
    xs = xs_ref[...]
    agg = dinv * (p0_ref[...] + p1_ref[...] + xs)
    dot = functools.partial(jnp.dot, preferred_element_type=jnp.float32)
    wzf = dot(wz_ref[...], lz_ref[...])
    whf = dot(wh_ref[...], lh_ref[...])
    z = jax.nn.sigmoid(dot(agg, wzf) + bz_ref[...])
    ht = jnp.tanh(dot(agg, whf) + bh_ref[...])
    h = (1.0 - z) * ht
    if final:
        out_ref[...] = dot(h, lw_ref[...]) + lb_ref[...]
    else:
        out_ref[...] = dinv * h


def _cell_call(final, deg_part, xs, p0, p1, p, lin_w, lin_b):
    row = pl.BlockSpec((BN, C), lambda i: (i, 0))
    full = pl.BlockSpec((C, C), lambda i: (0, 0))
    vec = pl.BlockSpec((1, C), lambda i: (0, 0))
    lz = p['lin_z_W'][:C]
    lh = p['lin_h_W'][:C]
    bz = (p['conv_z_b'] @ lz + p['lin_z_b'])[None, :]
    bh = (p['conv_h_b'] @ lh + p['lin_h_b'])[None, :]
    return pl.pallas_call(
        functools.partial(_cell_body, final),
        grid=(GRID,),
        in_specs=[pl.BlockSpec((1, NT, BN), lambda i: (i, 0, 0)), row, row, row,
                  full, full, vec, full, full, vec, full, vec],
        out_specs=row,
        out_shape=jax.ShapeDtypeStruct((N, C), jnp.float32),
    )(deg_part, xs, p0, p1,
      p['conv_z_W'], lz, bz, p['conv_h_W'], lh, bh,
      lin_w, lin_b[None, :])


def kernel(x, edge_index, edge_weight, params):
    src = edge_index[0].astype(jnp.int32)
    dst = edge_index[1].astype(jnp.int32)
    w = edge_weight.astype(jnp.float32)
    src3d = src.reshape(NT, CPT, K)
    dst3d = dst.reshape(NT, CPT, K)
    w3d = w.reshape(NT, CPT, K)

    deg_part = _deg_kernel(dst, w)
    deg_part = deg_part.reshape(NT, GRID, BN).transpose(1, 0, 2)
    xs1 = _prep_call(deg_part, x)
    parts1 = _scat_kernel(xs1, src3d, dst3d, w3d)
    zero_w = jnp.zeros((C, C), jnp.float32)
    zero_b = jnp.zeros((C,), jnp.float32)
    xs2 = _cell_call(False, deg_part, xs1, parts1[0], parts1[1],
                     params['tgcn1'], zero_w, zero_b)
    parts2 = _scat_kernel(xs2, src3d, dst3d, w3d)
    out = _cell_call(True, deg_part, xs2, parts2[0], parts2[1],
                     params['tgcn2'], params['lin_W'], params['lin_b'])
    return out
